# baseline (speedup 1.0000x reference)
import functools

import jax
import jax.numpy as jnp
from jax import lax
from jax.experimental import pallas as pl
from jax.experimental.pallas import tpu as pltpu
from jax.experimental.pallas import tpu_sc as plsc

N = 10000
E = 320000
H = 128
C = 64

NUM_CORES = 2
NUM_SUBCORES = 16
NW = NUM_CORES * NUM_SUBCORES
CH = 64
CPW = 160
EPAD = NW * CPW * CH
AGG_ROWS = NUM_SUBCORES * 640
ROWS_PC = AGG_ROWS // NUM_SUBCORES

NBUF = 4
BLK = 16
NBLK = CPW // BLK

_mesh = plsc.VectorSubcoreMesh(core_axis_name="c", subcore_axis_name="s")


@functools.partial(
    pl.kernel,
    mesh=_mesh,
    out_type=jax.ShapeDtypeStruct((NUM_CORES * AGG_ROWS, H), jnp.float32),
    scratch_types=[
        pltpu.VMEM((BLK, CH), jnp.int32),
        pltpu.VMEM((BLK, CH), jnp.int32),
        pltpu.VMEM((NBUF, CH, H), jnp.float32),
        pltpu.VMEM_SHARED((AGG_ROWS, H), jnp.float32),
    ] + [pltpu.SemaphoreType.DMA] * (2 * NBUF),
)
def _sc_agg(srcs_hbm, dsts_hbm, zeros_hbm, x_hbm, out_hbm,
            src_v, dstb_v, rows_v, agg_sh, *sems):
    gsem = sems[:NBUF]
    ssem = sems[NBUF:]
    c = lax.axis_index("c")
    s = lax.axis_index("s")
    wid = c * NUM_SUBCORES + s

    pltpu.sync_copy(zeros_hbm, rows_v.at[0])
    for k in range(ROWS_PC // CH):
        pltpu.sync_copy(rows_v.at[0],
                        agg_sh.at[pl.ds(s * ROWS_PC + k * CH, CH)])
    plsc.subcore_barrier()

    def wait_gather(b):
        pltpu.make_async_copy(
            x_hbm.at[pl.ds(0, CH)], rows_v.at[b], gsem[b]).wait()

    def wait_scatter(b):
        pltpu.make_async_copy(
            rows_v.at[b], agg_sh.at[pl.ds(0, CH)], ssem[b]).wait()

    def block_body(blk, carry):
        row0 = pl.multiple_of(wid * CPW + blk * BLK, 8)
        pltpu.sync_copy(srcs_hbm.at[pl.ds(row0, BLK)], src_v)
        pltpu.sync_copy(dsts_hbm.at[pl.ds(row0, BLK)], dstb_v)
        for b in range(NBUF):
            pltpu.async_copy(x_hbm.at[src_v.at[b]], rows_v.at[b], gsem[b])
        for i in range(BLK):
            b = i % NBUF
            wait_gather(b)
            pltpu.async_copy(rows_v.at[b], agg_sh.at[dstb_v.at[i]],
                             ssem[b], add=True)
            if i + NBUF < BLK:
                wait_scatter(b)
                pltpu.async_copy(x_hbm.at[src_v.at[i + NBUF]],
                                 rows_v.at[b], gsem[b])
        for b in range(NBUF):
            wait_scatter(b)
        return carry

    lax.fori_loop(0, NBLK, block_body, 0)
    plsc.subcore_barrier()

    pltpu.sync_copy(
        agg_sh.at[pl.ds(s * ROWS_PC, ROWS_PC)],
        out_hbm.at[pl.ds(c * AGG_ROWS + s * ROWS_PC, ROWS_PC)])


def _dense1_body(x_ref, p_ref, w1_ref, b1_ref, w2_ref, b2_ref,
                 g_ref, b_ref, o_ref):
    h = x_ref[...] + p_ref[0:N, :] + p_ref[AGG_ROWS:AGG_ROWS + N, :]
    a = jnp.maximum(
        jnp.dot(h, w1_ref[...], preferred_element_type=jnp.float32)
        + b1_ref[...], 0.0)
    a = jnp.dot(a, w2_ref[...], preferred_element_type=jnp.float32) + b2_ref[...]
    r = jnp.maximum(a, 0.0)
    mu = jnp.mean(r, axis=0, keepdims=True)
    var = jnp.mean((r - mu) ** 2, axis=0, keepdims=True)
    o_ref[...] = (r - mu) * lax.rsqrt(var + 1e-5) * g_ref[...] + b_ref[...]


def _dense2_body(x_ref, p_ref, w1_ref, b1_ref, w2_ref, b2_ref,
                 g_ref, b_ref, f1w_ref, f1b_ref, f2w_ref, f2b_ref, o_ref):
    h = x_ref[...] + p_ref[0:N, :] + p_ref[AGG_ROWS:AGG_ROWS + N, :]
    a = jnp.maximum(
        jnp.dot(h, w1_ref[...], preferred_element_type=jnp.float32)
        + b1_ref[...], 0.0)
    a = jnp.dot(a, w2_ref[...], preferred_element_type=jnp.float32) + b2_ref[...]
    r = jnp.maximum(a, 0.0)
    mu = jnp.mean(r, axis=0, keepdims=True)
    var = jnp.mean((r - mu) ** 2, axis=0, keepdims=True)
    x2 = (r - mu) * lax.rsqrt(var + 1e-5) * g_ref[...] + b_ref[...]
    y = jnp.maximum(
        jnp.dot(x2, f1w_ref[...], preferred_element_type=jnp.float32)
        + f1b_ref[...], 0.0)
    z = jnp.dot(y, f2w_ref[...], preferred_element_type=jnp.float32) + f2b_ref[...]
    m = jnp.max(z, axis=-1, keepdims=True)
    lse = jnp.log(jnp.sum(jnp.exp(z - m), axis=-1, keepdims=True)) + m
    o_ref[...] = z - lse


_dense1 = pl.pallas_call(
    _dense1_body,
    out_shape=jax.ShapeDtypeStruct((N, H), jnp.float32),
)

_dense2 = pl.pallas_call(
    _dense2_body,
    out_shape=jax.ShapeDtypeStruct((N, C), jnp.float32),
)


def kernel(features, edge_index, l0_w1, l0_b1, l0_w2, l0_b2, bn0_g, bn0_b,
           l1_w1, l1_b1, l1_w2, l1_b2, bn1_g, bn1_b, fc1_w, fc1_b, fc2_w, fc2_b):
    ei = edge_index.astype(jnp.int32)
    pad = EPAD - E
    pad_iota = jnp.arange(pad, dtype=jnp.int32)
    src = jnp.concatenate([ei[0], pad_iota % N]).reshape(-1, CH)
    dst = jnp.concatenate([ei[1], N + pad_iota % (AGG_ROWS - N)]).reshape(-1, CH)
    zeros_blk = jnp.zeros((CH, H), jnp.float32)

    b = lambda v: v.reshape(1, -1)

    p0 = _sc_agg(src, dst, zeros_blk, features)
    x1 = _dense1(features, p0, l0_w1, b(l0_b1), l0_w2, b(l0_b2),
                 b(bn0_g), b(bn0_b))
    p1 = _sc_agg(src, dst, zeros_blk, x1)
    out = _dense2(x1, p1, l1_w1, b(l1_b1), l1_w2, b(l1_b2),
                  b(bn1_g), b(bn1_b), fc1_w, b(fc1_b), fc2_w, b(fc2_b))
    return out


# speedup vs baseline: 1.0555x; 1.0555x over previous
import functools

import jax
import jax.numpy as jnp
from jax import lax
from jax.experimental import pallas as pl
from jax.experimental.pallas import tpu as pltpu
from jax.experimental.pallas import tpu_sc as plsc

N = 10000
E = 320000
H = 128
C = 64

NUM_CORES = 2
NUM_SUBCORES = 16
NW = NUM_CORES * NUM_SUBCORES
CH = 128
CPW = 80
EPAD = NW * CPW * CH
AGG_ROWS = NUM_SUBCORES * 640
ROWS_PC = AGG_ROWS // NUM_SUBCORES

NBUF = 2
BLK = 16
NBLK = CPW // BLK

_mesh = plsc.VectorSubcoreMesh(core_axis_name="c", subcore_axis_name="s")


@functools.partial(
    pl.kernel,
    mesh=_mesh,
    out_type=jax.ShapeDtypeStruct((NUM_CORES * AGG_ROWS, H), jnp.float32),
    scratch_types=[
        pltpu.VMEM((CPW, CH), jnp.int32),
        pltpu.VMEM((2, BLK, CH), jnp.int32),
        pltpu.VMEM((NBUF, CH, H), jnp.float32),
        pltpu.VMEM_SHARED((AGG_ROWS, H), jnp.float32),
    ] + [pltpu.SemaphoreType.DMA] * (2 * NBUF + 3),
)
def _sc_agg(srcs_hbm, dsts_hbm, zeros_hbm, x_hbm, out_hbm,
            src_v, dstb_v, rows_v, agg_sh, *sems):
    gsem = sems[:NBUF]
    ssem = sems[NBUF:2 * NBUF]
    srcsem = sems[2 * NBUF]
    dsem = sems[2 * NBUF + 1:2 * NBUF + 3]
    c = lax.axis_index("c")
    s = lax.axis_index("s")
    wid = c * NUM_SUBCORES + s
    row_base = wid * CPW

    pltpu.async_copy(srcs_hbm.at[pl.ds(row_base, CPW)], src_v, srcsem)
    pltpu.async_copy(dsts_hbm.at[pl.ds(row_base, BLK)], dstb_v.at[0], dsem[0])
    pltpu.sync_copy(zeros_hbm, rows_v.at[0])
    for k in range(ROWS_PC // CH):
        pltpu.sync_copy(rows_v.at[0],
                        agg_sh.at[pl.ds(s * ROWS_PC + k * CH, CH)])
    plsc.subcore_barrier()
    pltpu.make_async_copy(
        srcs_hbm.at[pl.ds(0, CPW)], src_v, srcsem).wait()

    def wait_gather(b):
        pltpu.make_async_copy(
            x_hbm.at[pl.ds(0, CH)], rows_v.at[b], gsem[b]).wait()

    def wait_scatter(b):
        pltpu.make_async_copy(
            rows_v.at[b], agg_sh.at[pl.ds(0, CH)], ssem[b]).wait()

    for blk in range(NBLK):
        par = blk % 2
        nxt = (blk + 1) % 2
        if blk + 1 < NBLK:
            pltpu.async_copy(
                dsts_hbm.at[pl.ds(row_base + (blk + 1) * BLK, BLK)],
                dstb_v.at[nxt], dsem[nxt])
        pltpu.make_async_copy(
            dsts_hbm.at[pl.ds(0, BLK)], dstb_v.at[par], dsem[par]).wait()
        for b in range(NBUF):
            pltpu.async_copy(x_hbm.at[src_v.at[blk * BLK + b]],
                             rows_v.at[b], gsem[b])
        for i in range(BLK):
            b = i % NBUF
            wait_gather(b)
            pltpu.async_copy(rows_v.at[b], agg_sh.at[dstb_v.at[par, i]],
                             ssem[b], add=True)
            if i + NBUF < BLK:
                wait_scatter(b)
                pltpu.async_copy(x_hbm.at[src_v.at[blk * BLK + i + NBUF]],
                                 rows_v.at[b], gsem[b])
        for b in range(NBUF):
            wait_scatter(b)
    plsc.subcore_barrier()

    pltpu.sync_copy(
        agg_sh.at[pl.ds(s * ROWS_PC, ROWS_PC)],
        out_hbm.at[pl.ds(c * AGG_ROWS + s * ROWS_PC, ROWS_PC)])


def _dense1_body(x_ref, p_ref, w1_ref, b1_ref, w2_ref, b2_ref,
                 g_ref, b_ref, o_ref):
    h = x_ref[...] + p_ref[0:N, :] + p_ref[AGG_ROWS:AGG_ROWS + N, :]
    a = jnp.maximum(
        jnp.dot(h, w1_ref[...], preferred_element_type=jnp.float32)
        + b1_ref[...], 0.0)
    a = jnp.dot(a, w2_ref[...], preferred_element_type=jnp.float32) + b2_ref[...]
    r = jnp.maximum(a, 0.0)
    mu = jnp.mean(r, axis=0, keepdims=True)
    var = jnp.mean(r * r, axis=0, keepdims=True) - mu * mu
    o_ref[...] = (r - mu) * lax.rsqrt(var + 1e-5) * g_ref[...] + b_ref[...]


def _dense2_body(x_ref, p_ref, w1_ref, b1_ref, w2_ref, b2_ref,
                 g_ref, b_ref, f1w_ref, f1b_ref, f2w_ref, f2b_ref, o_ref):
    h = x_ref[...] + p_ref[0:N, :] + p_ref[AGG_ROWS:AGG_ROWS + N, :]
    a = jnp.maximum(
        jnp.dot(h, w1_ref[...], preferred_element_type=jnp.float32)
        + b1_ref[...], 0.0)
    a = jnp.dot(a, w2_ref[...], preferred_element_type=jnp.float32) + b2_ref[...]
    r = jnp.maximum(a, 0.0)
    mu = jnp.mean(r, axis=0, keepdims=True)
    var = jnp.mean(r * r, axis=0, keepdims=True) - mu * mu
    x2 = (r - mu) * lax.rsqrt(var + 1e-5) * g_ref[...] + b_ref[...]
    y = jnp.maximum(
        jnp.dot(x2, f1w_ref[...], preferred_element_type=jnp.float32)
        + f1b_ref[...], 0.0)
    z = jnp.dot(y, f2w_ref[...], preferred_element_type=jnp.float32) + f2b_ref[...]
    m = jnp.max(z, axis=-1, keepdims=True)
    lse = jnp.log(jnp.sum(jnp.exp(z - m), axis=-1, keepdims=True)) + m
    o_ref[...] = z - lse


_dense1 = pl.pallas_call(
    _dense1_body,
    out_shape=jax.ShapeDtypeStruct((N, H), jnp.float32),
)

_dense2 = pl.pallas_call(
    _dense2_body,
    out_shape=jax.ShapeDtypeStruct((N, C), jnp.float32),
)


def kernel(features, edge_index, l0_w1, l0_b1, l0_w2, l0_b2, bn0_g, bn0_b,
           l1_w1, l1_b1, l1_w2, l1_b2, bn1_g, bn1_b, fc1_w, fc1_b, fc2_w, fc2_b):
    ei = edge_index.astype(jnp.int32)
    pad = EPAD - E
    pad_iota = jnp.arange(pad, dtype=jnp.int32)
    src = jnp.concatenate([ei[0], pad_iota % N]).reshape(-1, CH)
    dst = jnp.concatenate([ei[1], N + pad_iota % (AGG_ROWS - N)]).reshape(-1, CH)
    zeros_blk = jnp.zeros((CH, H), jnp.float32)

    b = lambda v: v.reshape(1, -1)

    p0 = _sc_agg(src, dst, zeros_blk, features)
    x1 = _dense1(features, p0, l0_w1, b(l0_b1), l0_w2, b(l0_b2),
                 b(bn0_g), b(bn0_b))
    p1 = _sc_agg(src, dst, zeros_blk, x1)
    out = _dense2(x1, p1, l1_w1, b(l1_b1), l1_w2, b(l1_b2),
                  b(bn1_g), b(bn1_b), fc1_w, b(fc1_b), fc2_w, b(fc2_b))
    return out
